# baseline (speedup 1.0000x reference)
import functools

import jax
import jax.numpy as jnp
from jax import lax
from jax.experimental import pallas as pl
from jax.experimental.pallas import tpu as pltpu
from jax.experimental.pallas import tpu_sc as plsc

N = 10000
D = 128
E = 320000

NCORES = 2
NSUB = 16
NWORK = NCORES * NSUB
EPW = E // NWORK
W = 80
NWIN = EPW // W
NPAIR = (NWIN + 1) // 2
STRIPE = 624
TAIL = N - NSUB * STRIPE
NCPAD = 10240
CSTRIPE = NCPAD // NSUB


NB = 4


def _sc_agg_body(with_cnt, h_hbm, src_hbm, dst_hbm, z2_hbm, z1_hbm,
                 ones_hbm, acc_out, cnt_out,
                 rows0, rows1, rows2, rows3,
                 swin0, swin1, swin2, swin3,
                 dwin0, dwin1, dwin2, dwin3, ones_v, acc_s, cnt_s,
                 gsem0, gsem1, gsem2, gsem3,
                 ssem0, ssem1, ssem2, ssem3,
                 csem0, csem1, csem2, csem3,
                 isem0, isem1, isem2, isem3):
    c = lax.axis_index("c")
    s = lax.axis_index("s")
    wid = s * NCORES + c
    rows = (rows0, rows1, rows2, rows3)
    swin = (swin0, swin1, swin2, swin3)
    dwin = (dwin0, dwin1, dwin2, dwin3)
    gsem = (gsem0, gsem1, gsem2, gsem3)
    ssem = (ssem0, ssem1, ssem2, ssem3)
    csem = (csem0, csem1, csem2, csem3)
    isem = (isem0, isem1, isem2, isem3)

    r0 = pl.multiple_of(s * STRIPE, 8)
    c0 = pl.multiple_of(s * CSTRIPE, 128)
    pltpu.sync_copy(z2_hbm.at[pl.ds(r0, STRIPE)], acc_s.at[pl.ds(r0, STRIPE)])
    if with_cnt:
        pltpu.sync_copy(z1_hbm.at[pl.ds(c0, CSTRIPE)],
                        cnt_s.at[pl.ds(c0, CSTRIPE)])
        pltpu.sync_copy(ones_hbm, ones_v)

    @pl.when(s == 0)
    def _zero_tail():
        pltpu.sync_copy(z2_hbm.at[pl.ds(NSUB * STRIPE, TAIL)],
                        acc_s.at[pl.ds(NSUB * STRIPE, TAIL)])

    plsc.subcore_barrier()
    e0 = pl.multiple_of(wid * EPW, 16)

    def src_window(w):
        return src_hbm.at[pl.ds(e0 + pl.multiple_of(w * W, 16), W)]

    def dst_window(w):
        return dst_hbm.at[pl.ds(e0 + pl.multiple_of(w * W, 16), W)]

    def fetch_idx(w, q):
        pltpu.async_copy(src_window(w), swin[q], isem[q])
        pltpu.async_copy(dst_window(w), dwin[q], isem[q])

    def wait_idx(w, q):
        pltpu.make_async_copy(src_window(w), swin[q], isem[q]).wait()
        pltpu.make_async_copy(dst_window(w), dwin[q], isem[q]).wait()

    def start_gather(w, q):
        pltpu.async_copy(h_hbm.at[swin[q]], rows[q], gsem[q])

    def wait_scatter(w, q):
        pltpu.make_async_copy(rows[q], acc_s.at[dwin[q]], ssem[q]).wait()
        if with_cnt:
            pltpu.make_async_copy(ones_v, cnt_s.at[dwin[q]],
                                  csem[q]).wait()

    fetch_idx(0, 0)
    fetch_idx(1, 1)
    wait_idx(0, 0)
    start_gather(0, 0)

    def stage(j, carry):
        for b in range(NB):
            w = NB * j + b

            @pl.when(w < NWIN)
            def _window():
                @pl.when(w >= 2)
                def _retire():
                    wait_scatter(w - 2, (b - 2) % NB)

                @pl.when(w + 2 < NWIN)
                def _prefetch():
                    fetch_idx(w + 2, (b + 2) % NB)

                pltpu.make_async_copy(h_hbm.at[swin[b]], rows[b],
                                      gsem[b]).wait()

                pltpu.async_copy(rows[b], acc_s.at[dwin[b]], ssem[b],
                                 add=True)
                if with_cnt:
                    pltpu.async_copy(ones_v, cnt_s.at[dwin[b]],
                                     csem[b], add=True)

                @pl.when(w + 1 < NWIN)
                def _next_gather():
                    wait_idx(w + 1, (b + 1) % NB)
                    start_gather(w + 1, (b + 1) % NB)
        return carry

    lax.fori_loop(0, (NWIN + NB - 1) // NB, stage, 0)
    wait_scatter(NWIN - 2, (NWIN - 2) % NB)
    wait_scatter(NWIN - 1, (NWIN - 1) % NB)
    plsc.subcore_barrier()

    cbase = pl.multiple_of(c * NCPAD, 128)
    pltpu.sync_copy(acc_s.at[pl.ds(r0, STRIPE)],
                    acc_out.at[c, pl.ds(r0, STRIPE)])
    if with_cnt:
        pltpu.sync_copy(cnt_s.at[pl.ds(c0, CSTRIPE)],
                        cnt_out.at[pl.ds(cbase + c0, CSTRIPE)])

    @pl.when(s == 0)
    def _write_tail():
        pltpu.sync_copy(acc_s.at[pl.ds(NSUB * STRIPE, TAIL)],
                        acc_out.at[c, pl.ds(NSUB * STRIPE, TAIL)])


def _make_sc_agg(with_cnt):
    return pl.kernel(
        functools.partial(_sc_agg_body, with_cnt),
        out_type=[
            jax.ShapeDtypeStruct((NCORES, N, D), jnp.float32),
            jax.ShapeDtypeStruct((NCORES * NCPAD,), jnp.float32),
        ],
        mesh=plsc.VectorSubcoreMesh(core_axis_name="c", subcore_axis_name="s"),
        scratch_types=(
            [pltpu.VMEM((W, D), jnp.float32)] * NB
            + [pltpu.VMEM((W,), jnp.int32)] * (2 * NB)
            + [pltpu.VMEM((W,), jnp.float32)]
            + [pltpu.VMEM_SHARED((N, D), jnp.float32),
               pltpu.VMEM_SHARED((NCPAD,), jnp.float32)]
            + [pltpu.SemaphoreType.DMA] * (4 * NB)
        ),
    )


_sc_agg_cnt = _make_sc_agg(True)
_sc_agg_nocnt = _make_sc_agg(False)


def _tc_body(h_ref, acc_ref, invb_ref, wlT_ref, wrT_ref, bl_ref, mask_ref,
             out_ref, *, apply_mask):
    mean = (acc_ref[0] + acc_ref[1]) * invb_ref[...]
    out = (jnp.dot(mean, wlT_ref[...], preferred_element_type=jnp.float32)
           + jnp.dot(h_ref[...], wrT_ref[...], preferred_element_type=jnp.float32)
           + bl_ref[...])
    if apply_mask:
        out = jnp.maximum(out * mask_ref[...], 0.0)
    out_ref[...] = out


RB = 1000


def _tc_layer(h, acc, invb, wlT, wrT, bl2d, mask, apply_mask):
    grid = (N // RB,)
    return pl.pallas_call(
        functools.partial(_tc_body, apply_mask=apply_mask),
        grid=grid,
        in_specs=[
            pl.BlockSpec((RB, D), lambda i: (i, 0)),
            pl.BlockSpec((NCORES, RB, D), lambda i: (0, i, 0)),
            pl.BlockSpec((RB, D), lambda i: (i, 0)),
            pl.BlockSpec((D, D), lambda i: (0, 0)),
            pl.BlockSpec((D, D), lambda i: (0, 0)),
            pl.BlockSpec((1, D), lambda i: (0, 0)),
            pl.BlockSpec((RB, D), lambda i: (i, 0)),
        ],
        out_specs=pl.BlockSpec((RB, D), lambda i: (i, 0)),
        out_shape=jax.ShapeDtypeStruct((N, D), jnp.float32),
    )(h, acc, invb, wlT, wrT, bl2d, mask)


def kernel(x, edge_index, edge_idx_1_1, Wl1, bl1, Wr1, Wl2, bl2, Wr2,
           Wl3, bl3, Wr3):
    f32 = jnp.float32
    z2 = jnp.zeros((N, D), f32)
    z1 = jnp.zeros((NCPAD,), f32)
    ones_w = jnp.ones((W,), f32)

    src_a, dst_a = edge_index[0], edge_index[1]
    src_b, dst_b = edge_idx_1_1[0], edge_idx_1_1[1]

    keep1 = jax.random.bernoulli(jax.random.key(1), 0.5, (N, D))
    keep2 = jax.random.bernoulli(jax.random.key(2), 0.5, (N, D))
    mask1 = keep1.astype(f32) * 2.0
    mask2 = keep2.astype(f32) * 2.0

    def layer(h, src, dst, Wl, bl, Wr, mask, apply_mask, inv=None):
        if inv is None:
            acc, cnt = _sc_agg_cnt(h, src, dst, z2, z1, ones_w)
            cnt = cnt.reshape(NCORES, NCPAD)[:, :N]
            inv = 1.0 / jnp.maximum(cnt[0] + cnt[1], 1.0)
        else:
            acc, _ = _sc_agg_nocnt(h, src, dst, z2, z1, ones_w)
        invb = jnp.broadcast_to(inv[:, None], (N, D))
        out = _tc_layer(h, acc, invb, Wl.T, Wr.T, bl[None, :], mask,
                        apply_mask)
        return out, inv

    h, inv_a = layer(x, src_a, dst_a, Wl1, bl1, Wr1, mask1, True)
    h, _ = layer(h, src_b, dst_b, Wl2, bl2, Wr2, mask2, True)
    h, _ = layer(h, src_a, dst_a, Wl3, bl3, Wr3, mask1, False,
                 inv=inv_a)
    return h


# speedup vs baseline: 1.4315x; 1.4315x over previous
import functools

import jax
import jax.numpy as jnp
from jax import lax
from jax.experimental import pallas as pl
from jax.experimental.pallas import tpu as pltpu
from jax.experimental.pallas import tpu_sc as plsc

N = 10000
D = 128
E = 320000

NCORES = 2
NSUB = 16
NWORK = NCORES * NSUB
EPW = E // NWORK
W = 80
NWIN = EPW // W
NPAIR = (NWIN + 1) // 2
STRIPE = 624
TAIL = N - NSUB * STRIPE
NCPAD = 10240
CSTRIPE = NCPAD // NSUB


NB = 3


def _sc_agg_body(with_cnt, h_hbm, src_hbm, dst_hbm, z2_hbm, z1_hbm,
                 ones_hbm, acc_out, cnt_out, src_v,
                 rows0, rows1, rows2,
                 dwin0, dwin1, dwin2, ones_v, acc_s, cnt_s,
                 gsem0, gsem1, gsem2,
                 ssem0, ssem1, ssem2,
                 csem0, csem1, csem2,
                 isem0, isem1, isem2):
    c = lax.axis_index("c")
    s = lax.axis_index("s")
    wid = s * NCORES + c
    rows = (rows0, rows1, rows2)
    dwin = (dwin0, dwin1, dwin2)
    gsem = (gsem0, gsem1, gsem2)
    ssem = (ssem0, ssem1, ssem2)
    csem = (csem0, csem1, csem2)
    isem = (isem0, isem1, isem2)

    r0 = pl.multiple_of(s * STRIPE, 8)
    c0 = pl.multiple_of(s * CSTRIPE, 128)
    pltpu.sync_copy(z2_hbm.at[pl.ds(r0, STRIPE)], acc_s.at[pl.ds(r0, STRIPE)])
    if with_cnt:
        pltpu.sync_copy(z1_hbm.at[pl.ds(c0, CSTRIPE)],
                        cnt_s.at[pl.ds(c0, CSTRIPE)])
        pltpu.sync_copy(ones_hbm, ones_v)

    @pl.when(s == 0)
    def _zero_tail():
        pltpu.sync_copy(z2_hbm.at[pl.ds(NSUB * STRIPE, TAIL)],
                        acc_s.at[pl.ds(NSUB * STRIPE, TAIL)])

    e0 = pl.multiple_of(wid * EPW, 16)
    pltpu.sync_copy(src_hbm.at[pl.ds(e0, EPW)], src_v)
    plsc.subcore_barrier()

    def src_slice(w):
        return src_v.at[pl.ds(pl.multiple_of(w * W, 16), W)]

    def dst_window(w):
        return dst_hbm.at[pl.ds(e0 + pl.multiple_of(w * W, 16), W)]

    def fetch_idx(w, q):
        pltpu.async_copy(dst_window(w), dwin[q], isem[q])

    def wait_idx(w, q):
        pltpu.make_async_copy(dst_window(w), dwin[q], isem[q]).wait()

    def start_gather(w, q):
        pltpu.async_copy(h_hbm.at[src_slice(w)], rows[q], gsem[q])

    def wait_scatter(w, q):
        pltpu.make_async_copy(rows[q], acc_s.at[dwin[q]], ssem[q]).wait()
        if with_cnt:
            pltpu.make_async_copy(ones_v, cnt_s.at[dwin[q]],
                                  csem[q]).wait()

    fetch_idx(0, 0)
    fetch_idx(1, 1)
    start_gather(0, 0)
    start_gather(1, 1)

    def stage(j, carry):
        for b in range(NB):
            w = NB * j + b

            @pl.when(w < NWIN)
            def _window():
                @pl.when(w >= 1)
                def _retire():
                    wait_scatter(w - 1, (b + NB - 1) % NB)

                @pl.when(w + 2 < NWIN)
                def _prefetch():
                    fetch_idx(w + 2, (b + 2) % NB)

                pltpu.make_async_copy(h_hbm.at[src_slice(w)], rows[b],
                                      gsem[b]).wait()
                wait_idx(w, b)

                pltpu.async_copy(rows[b], acc_s.at[dwin[b]], ssem[b],
                                 add=True)
                if with_cnt:
                    pltpu.async_copy(ones_v, cnt_s.at[dwin[b]],
                                     csem[b], add=True)

                @pl.when(w + 2 < NWIN)
                def _next_gather():
                    start_gather(w + 2, (b + 2) % NB)
        return carry

    lax.fori_loop(0, (NWIN + NB - 1) // NB, stage, 0)
    wait_scatter(NWIN - 1, (NWIN - 1) % NB)
    plsc.subcore_barrier()

    cbase = pl.multiple_of(c * NCPAD, 128)
    pltpu.sync_copy(acc_s.at[pl.ds(r0, STRIPE)],
                    acc_out.at[c, pl.ds(r0, STRIPE)])
    if with_cnt:
        pltpu.sync_copy(cnt_s.at[pl.ds(c0, CSTRIPE)],
                        cnt_out.at[pl.ds(cbase + c0, CSTRIPE)])

    @pl.when(s == 0)
    def _write_tail():
        pltpu.sync_copy(acc_s.at[pl.ds(NSUB * STRIPE, TAIL)],
                        acc_out.at[c, pl.ds(NSUB * STRIPE, TAIL)])


def _make_sc_agg(with_cnt):
    return pl.kernel(
        functools.partial(_sc_agg_body, with_cnt),
        out_type=[
            jax.ShapeDtypeStruct((NCORES, N, D), jnp.float32),
            jax.ShapeDtypeStruct((NCORES * NCPAD,), jnp.float32),
        ],
        mesh=plsc.VectorSubcoreMesh(core_axis_name="c", subcore_axis_name="s"),
        scratch_types=(
            [pltpu.VMEM((EPW,), jnp.int32)]
            + [pltpu.VMEM((W, D), jnp.float32)] * NB
            + [pltpu.VMEM((W,), jnp.int32)] * NB
            + [pltpu.VMEM((W,), jnp.float32)]
            + [pltpu.VMEM_SHARED((N, D), jnp.float32),
               pltpu.VMEM_SHARED((NCPAD,), jnp.float32)]
            + [pltpu.SemaphoreType.DMA] * (4 * NB)
        ),
    )


_sc_agg_cnt = _make_sc_agg(True)
_sc_agg_nocnt = _make_sc_agg(False)


def _tc_body(h_ref, acc_ref, invb_ref, wlT_ref, wrT_ref, bl_ref, mask_ref,
             out_ref, *, apply_mask):
    mean = (acc_ref[0] + acc_ref[1]) * invb_ref[...]
    out = (jnp.dot(mean, wlT_ref[...], preferred_element_type=jnp.float32)
           + jnp.dot(h_ref[...], wrT_ref[...], preferred_element_type=jnp.float32)
           + bl_ref[...])
    if apply_mask:
        out = jnp.maximum(out * mask_ref[...], 0.0)
    out_ref[...] = out


RB = 1000


def _tc_layer(h, acc, invb, wlT, wrT, bl2d, mask, apply_mask):
    grid = (N // RB,)
    return pl.pallas_call(
        functools.partial(_tc_body, apply_mask=apply_mask),
        grid=grid,
        in_specs=[
            pl.BlockSpec((RB, D), lambda i: (i, 0)),
            pl.BlockSpec((NCORES, RB, D), lambda i: (0, i, 0)),
            pl.BlockSpec((RB, D), lambda i: (i, 0)),
            pl.BlockSpec((D, D), lambda i: (0, 0)),
            pl.BlockSpec((D, D), lambda i: (0, 0)),
            pl.BlockSpec((1, D), lambda i: (0, 0)),
            pl.BlockSpec((RB, D), lambda i: (i, 0)),
        ],
        out_specs=pl.BlockSpec((RB, D), lambda i: (i, 0)),
        out_shape=jax.ShapeDtypeStruct((N, D), jnp.float32),
    )(h, acc, invb, wlT, wrT, bl2d, mask)


def kernel(x, edge_index, edge_idx_1_1, Wl1, bl1, Wr1, Wl2, bl2, Wr2,
           Wl3, bl3, Wr3):
    f32 = jnp.float32
    z2 = jnp.zeros((N, D), f32)
    z1 = jnp.zeros((NCPAD,), f32)
    ones_w = jnp.ones((W,), f32)

    src_a, dst_a = edge_index[0], edge_index[1]
    src_b, dst_b = edge_idx_1_1[0], edge_idx_1_1[1]

    keep1 = jax.random.bernoulli(jax.random.key(1), 0.5, (N, D))
    keep2 = jax.random.bernoulli(jax.random.key(2), 0.5, (N, D))
    mask1 = keep1.astype(f32) * 2.0
    mask2 = keep2.astype(f32) * 2.0

    def layer(h, src, dst, Wl, bl, Wr, mask, apply_mask, inv=None):
        if inv is None:
            acc, cnt = _sc_agg_cnt(h, src, dst, z2, z1, ones_w)
            cnt = cnt.reshape(NCORES, NCPAD)[:, :N]
            inv = 1.0 / jnp.maximum(cnt[0] + cnt[1], 1.0)
        else:
            acc, _ = _sc_agg_nocnt(h, src, dst, z2, z1, ones_w)
        invb = jnp.broadcast_to(inv[:, None], (N, D))
        out = _tc_layer(h, acc, invb, Wl.T, Wr.T, bl[None, :], mask,
                        apply_mask)
        return out, inv

    h, inv_a = layer(x, src_a, dst_a, Wl1, bl1, Wr1, mask1, True)
    h, _ = layer(h, src_b, dst_b, Wl2, bl2, Wr2, mask2, True)
    h, _ = layer(h, src_a, dst_a, Wl3, bl3, Wr3, mask1, False,
                 inv=inv_a)
    return h
